# baseline (speedup 1.0000x reference)
import functools

import jax
import jax.numpy as jnp
from jax import lax
from jax.experimental import pallas as pl
from jax.experimental.pallas import tpu as pltpu
from jax.experimental.pallas import tpu_sc as plsc

NUM_STOCKS = 100000
CELL_SIZE = 64
BATCH = 16384

NC, NS = 2, 16
NW = NC * NS
B_PER_W = BATCH // NW
CHUNK = 128
NCHUNK = B_PER_W // CHUNK


def _encoder_gather(idx_flat, packed):
    mesh = plsc.VectorSubcoreMesh(core_axis_name="c", subcore_axis_name="s")
    out_t = jax.ShapeDtypeStruct((BATCH, 2 * CELL_SIZE), jnp.float32)

    @functools.partial(
        pl.kernel,
        out_type=out_t,
        mesh=mesh,
        scratch_types=[
            pltpu.VMEM((B_PER_W,), jnp.int32),
            pltpu.VMEM((B_PER_W, 2 * CELL_SIZE), jnp.float32),
            pltpu.SemaphoreType.DMA,
            pltpu.SemaphoreType.DMA,
        ],
    )
    def k(tab_hbm, idx_hbm, o_hbm, idx_v, rows_v, sem_g, sem_w):
        wid = lax.axis_index("s") * NC + lax.axis_index("c")
        base = wid * B_PER_W
        pltpu.sync_copy(idx_hbm.at[pl.ds(base, B_PER_W)], idx_v)

        gathers = []
        for j in range(NCHUNK):
            sl = pl.ds(j * CHUNK, CHUNK)
            gathers.append(pltpu.async_copy(
                tab_hbm.at[idx_v.at[sl]], rows_v.at[sl], sem_g))
        for c in gathers:
            c.wait()
        pltpu.async_copy(rows_v, o_hbm.at[pl.ds(base, B_PER_W)], sem_w).wait()

    return k(packed, idx_flat)


def kernel(Stock_ID, emb0, emb1):
    idx_flat = Stock_ID.reshape(BATCH).astype(jnp.int32)
    packed = jnp.concatenate([emb0, emb1], axis=1)
    out = _encoder_gather(idx_flat, packed)
    return (out[:, :CELL_SIZE], out[:, CELL_SIZE:])
